# Initial kernel scaffold; baseline (speedup 1.0000x reference)
import jax
import jax.numpy as jnp
from jax.experimental import pallas as pl


def kernel(x_pfc, edge_index, W_lin, W_src, W_dst, W_pos, b_pos):
    raise NotImplementedError("write your pallas kernel here")


# speedup vs baseline: 45.7264x; 45.7264x over previous
import functools

import jax
import jax.numpy as jnp
from jax import lax
from jax.experimental import pallas as pl
from jax.experimental.pallas import tpu as pltpu
from jax.experimental.pallas import tpu_sc as plsc

NC = 2
NS = 16
NW = NC * NS
B = 1024
SUB = B // 128
TW = 16
AW = 8
GRID = 32


def _prologue_body(x_ref, w_ref, b_ref, t_ref):
    t = jnp.dot(x_ref[...], w_ref[...], preferred_element_type=jnp.float32)
    t_ref[...] = t + b_ref[...]


def _epilogue_body(t_ref, acc_ref, o_ref):
    t = t_ref[...]
    a = acc_ref[...]
    den = 1.0 + a[0, :, 0:2] + a[1, :, 0:2] + 1e-16
    num = t[:, 2:4] + t[:, 4:6] + a[0, :, 2:4] + a[1, :, 2:4]
    o_ref[...] = num / den


def _sc_edge_body(npad, epad, ndum,
                  t_hbm, s_hbm, d_hbm, z_hbm, acc_out,
                  sbuf, dbuf, dibuf, gs, gd, sv, acc, sem_s, sem_d):
    cid = lax.axis_index("c")
    sid = lax.axis_index("s")
    wid = cid * NS + sid
    rows_per = npad // NS
    r0 = sid * rows_per

    pltpu.sync_copy(z_hbm.at[pl.ds(r0, rows_per)], acc.at[pl.ds(r0, rows_per)])
    pltpu.sync_copy(z_hbm.at[pl.ds(0, B)], sv)
    plsc.subcore_barrier()

    ew = epad // NW
    nchunks = ew // B
    lane = lax.iota(jnp.int32, 16)
    cols = [jnp.full((16,), k, jnp.int32) for k in range(6)]

    def chunk(i, carry):
        off = wid * ew + i * B
        for j in range(SUB):
            pltpu.sync_copy(s_hbm.at[pl.ds(off + j * 128, 128)], sbuf.at[j])
            pltpu.sync_copy(d_hbm.at[pl.ds(off + j * 128, 128)], dbuf.at[j])
        cps = []
        for j in range(SUB):
            sl = pl.ds(j * 128, 128)
            cps.append(pltpu.async_copy(t_hbm.at[sbuf.at[j]], gs.at[sl], sem_s))
            cps.append(pltpu.async_copy(t_hbm.at[dbuf.at[j]], gd.at[sl], sem_d))
        for cp in cps:
            cp.wait()

        def grp(j, c2):
            base = j * 16
            rows = base + lane
            svec = sbuf[j >> 3, pl.ds((j & 7) * 16, 16)]
            dvec = dbuf[j >> 3, pl.ds((j & 7) * 16, 16)]
            di = jnp.where(svec == dvec, ndum, dvec)
            dibuf[j >> 3, pl.ds((j & 7) * 16, 16)] = di
            us0 = plsc.load_gather(gs, [rows, cols[0]])
            us1 = plsc.load_gather(gs, [rows, cols[1]])
            vs0 = plsc.load_gather(gs, [rows, cols[2]])
            vs1 = plsc.load_gather(gs, [rows, cols[3]])
            ud0 = plsc.load_gather(gd, [rows, cols[0]])
            ud1 = plsc.load_gather(gd, [rows, cols[1]])
            qd0 = plsc.load_gather(gd, [rows, cols[4]])
            qd1 = plsc.load_gather(gd, [rows, cols[5]])
            ex0 = jnp.exp(ud0 - us0)
            ex1 = jnp.exp(ud1 - us1)
            va0 = vs0 + qd0
            va1 = vs1 + qd1
            plsc.store_scatter(sv, [rows, cols[0]], ex0)
            plsc.store_scatter(sv, [rows, cols[1]], ex1)
            plsc.store_scatter(sv, [rows, cols[2]], ex0 * va0)
            plsc.store_scatter(sv, [rows, cols[3]], ex1 * va1)
            return c2

        lax.fori_loop(0, B // 16, grp, 0)
        for j in range(SUB):
            sl = pl.ds(j * 128, 128)
            pltpu.sync_copy(sv.at[sl], acc.at[dibuf.at[j]], add=True)
        return carry

    lax.fori_loop(0, nchunks, chunk, 0)
    plsc.subcore_barrier()
    pltpu.sync_copy(acc.at[pl.ds(r0, rows_per)],
                    acc_out.at[cid, pl.ds(r0, rows_per)])


def kernel(x_pfc, edge_index, W_lin, W_src, W_dst, W_pos, b_pos):
    n, d_in = x_pfc.shape
    e = edge_index.shape[1]
    d_pos = W_pos.shape[0]

    npad = ((n + 1 + NS * 8 - 1) // (NS * 8)) * (NS * 8)
    ch = NW * B
    epad = ((e + ch - 1) // ch) * ch
    ndum = n

    wq = jnp.zeros((d_in, 2), jnp.float32).at[:d_pos, :].set(W_pos)
    wu = W_src + wq
    wv = W_lin - wq
    w16 = jnp.concatenate(
        [wu, wv, wq, jnp.zeros((d_in, TW - 6), jnp.float32)], axis=1)
    b16 = jnp.concatenate([jnp.zeros((2,), jnp.float32), b_pos,
                           jnp.zeros((TW - 4,), jnp.float32)]).reshape(1, TW)
    x_pad = jnp.pad(x_pfc, ((0, npad - n), (0, 0)))
    srcs = edge_index[1]
    dsts = edge_index[0]
    if epad != e:
        pad = jnp.zeros((epad - e,), jnp.int32)
        srcs = jnp.concatenate([srcs, pad])
        dsts = jnp.concatenate([dsts, pad])
    zeros_acc = jnp.zeros((npad, AW), jnp.float32)

    bn = npad // GRID

    t16 = pl.pallas_call(
        _prologue_body,
        grid=(GRID,),
        in_specs=[
            pl.BlockSpec((bn, d_in), lambda i: (i, 0)),
            pl.BlockSpec((d_in, TW), lambda i: (0, 0)),
            pl.BlockSpec((1, TW), lambda i: (0, 0)),
        ],
        out_specs=pl.BlockSpec((bn, TW), lambda i: (i, 0)),
        out_shape=jax.ShapeDtypeStruct((npad, TW), jnp.float32),
    )(x_pad, w16, b16)

    mesh = plsc.VectorSubcoreMesh(core_axis_name="c", subcore_axis_name="s")
    sc_fn = pl.kernel(
        functools.partial(_sc_edge_body, npad, epad, ndum),
        out_type=jax.ShapeDtypeStruct((NC, npad, AW), jnp.float32),
        mesh=mesh,
        scratch_types=[
            pltpu.VMEM((SUB, 128), jnp.int32),
            pltpu.VMEM((SUB, 128), jnp.int32),
            pltpu.VMEM((SUB, 128), jnp.int32),
            pltpu.VMEM((B, TW), jnp.float32),
            pltpu.VMEM((B, TW), jnp.float32),
            pltpu.VMEM((B, AW), jnp.float32),
            pltpu.VMEM_SHARED((npad, AW), jnp.float32),
            pltpu.SemaphoreType.DMA,
            pltpu.SemaphoreType.DMA,
        ],
        compiler_params=pltpu.CompilerParams(
            use_tc_tiling_on_sc=False, needs_layout_passes=False),
    )
    acc = sc_fn(t16, srcs, dsts, zeros_acc)

    out = pl.pallas_call(
        _epilogue_body,
        grid=(GRID,),
        in_specs=[
            pl.BlockSpec((bn, TW), lambda i: (i, 0)),
            pl.BlockSpec((NC, bn, AW), lambda i: (0, i, 0)),
        ],
        out_specs=pl.BlockSpec((bn, 2), lambda i: (i, 0)),
        out_shape=jax.ShapeDtypeStruct((n, 2), jnp.float32),
    )(t16, acc)
    return out


# speedup vs baseline: 82.6564x; 1.8076x over previous
import functools

import jax
import jax.numpy as jnp
from jax import lax
from jax.experimental import pallas as pl
from jax.experimental.pallas import tpu as pltpu
from jax.experimental.pallas import tpu_sc as plsc

NC = 2
NS = 16
NW = NC * NS
B = 1024
SUB = B // 128
TW = 16
AW = 8
GRID = 32


def _prologue_body(x_ref, w_ref, b_ref, t_ref):
    t = jnp.dot(x_ref[...], w_ref[...], preferred_element_type=jnp.float32)
    t_ref[...] = t + b_ref[...]


def _epilogue_body(t_ref, acc_ref, o_ref):
    t = t_ref[...]
    a = acc_ref[...]
    den = 1.0 + a[0, :, 0:2] + a[1, :, 0:2] + 1e-16
    num = t[:, 2:4] + t[:, 4:6] + a[0, :, 2:4] + a[1, :, 2:4]
    o_ref[...] = num / den


def _sc_edge_body(npad, epad, ndum,
                  t_hbm, s_hbm, d_hbm, z_hbm, acc_out,
                  sbuf, dbuf, dibuf, gs, gd, sv, acc,
                  sem_s, sem_d, sem_i, sem_v):
    cid = lax.axis_index("c")
    sid = lax.axis_index("s")
    wid = cid * NS + sid
    rows_per = npad // NS
    r0 = sid * rows_per

    pltpu.sync_copy(z_hbm.at[pl.ds(r0, rows_per)], acc.at[pl.ds(r0, rows_per)])
    pltpu.sync_copy(z_hbm.at[pl.ds(0, B)], sv)
    plsc.subcore_barrier()

    ew = epad // NW
    nchunks = ew // B
    irows = ew // 128
    ir0 = wid * irows
    lane = lax.iota(jnp.int32, 16)
    cols = [jnp.full((16,), k, jnp.int32) for k in range(6)]

    pltpu.sync_copy(s_hbm.at[pl.ds(ir0, SUB)], sbuf)
    pltpu.sync_copy(d_hbm.at[pl.ds(ir0, SUB)], dbuf)

    def chunk(i, carry):
        cps = []
        for j in range(SUB):
            sl = pl.ds(j * 128, 128)
            cps.append(pltpu.async_copy(t_hbm.at[sbuf.at[j]], gs.at[sl], sem_s))
            cps.append(pltpu.async_copy(t_hbm.at[dbuf.at[j]], gd.at[sl], sem_d))

        @pl.when(i > 0)
        def _():
            for j in range(SUB):
                sl = pl.ds(j * 128, 128)
                pltpu.make_async_copy(
                    sv.at[sl], acc.at[dibuf.at[j]], sem_v).wait()

        def grp_di(j, c2):
            svec = sbuf[j >> 3, pl.ds((j & 7) * 16, 16)]
            dvec = dbuf[j >> 3, pl.ds((j & 7) * 16, 16)]
            di = jnp.where(svec == dvec, ndum, dvec)
            dibuf[j >> 3, pl.ds((j & 7) * 16, 16)] = di
            return c2

        lax.fori_loop(0, B // 16, grp_di, 0)

        for cp in cps:
            cp.wait()

        @pl.when(i + 1 < nchunks)
        def _():
            nxt = ir0 + (i + 1) * SUB
            pltpu.async_copy(s_hbm.at[pl.ds(nxt, SUB)], sbuf, sem_i)
            pltpu.async_copy(d_hbm.at[pl.ds(nxt, SUB)], dbuf, sem_i)

        def grp(j, c2):
            base = j * 16
            rows = base + lane
            us0 = plsc.load_gather(gs, [rows, cols[0]])
            us1 = plsc.load_gather(gs, [rows, cols[1]])
            vs0 = plsc.load_gather(gs, [rows, cols[2]])
            vs1 = plsc.load_gather(gs, [rows, cols[3]])
            ud0 = plsc.load_gather(gd, [rows, cols[0]])
            ud1 = plsc.load_gather(gd, [rows, cols[1]])
            qd0 = plsc.load_gather(gd, [rows, cols[4]])
            qd1 = plsc.load_gather(gd, [rows, cols[5]])
            ex0 = jnp.exp(ud0 - us0)
            ex1 = jnp.exp(ud1 - us1)
            va0 = vs0 + qd0
            va1 = vs1 + qd1
            plsc.store_scatter(sv, [rows, cols[0]], ex0)
            plsc.store_scatter(sv, [rows, cols[1]], ex1)
            plsc.store_scatter(sv, [rows, cols[2]], ex0 * va0)
            plsc.store_scatter(sv, [rows, cols[3]], ex1 * va1)
            return c2

        lax.fori_loop(0, B // 16, grp, 0)

        for j in range(SUB):
            sl = pl.ds(j * 128, 128)
            pltpu.async_copy(sv.at[sl], acc.at[dibuf.at[j]], sem_v, add=True)

        @pl.when(i + 1 < nchunks)
        def _():
            pltpu.make_async_copy(s_hbm.at[pl.ds(ir0, SUB)], sbuf, sem_i).wait()
            pltpu.make_async_copy(d_hbm.at[pl.ds(ir0, SUB)], dbuf, sem_i).wait()
        return carry

    lax.fori_loop(0, nchunks, chunk, 0)

    for j in range(SUB):
        sl = pl.ds(j * 128, 128)
        pltpu.make_async_copy(sv.at[sl], acc.at[dibuf.at[j]], sem_v).wait()
    plsc.subcore_barrier()
    pltpu.sync_copy(acc.at[pl.ds(r0, rows_per)],
                    acc_out.at[cid, pl.ds(r0, rows_per)])


def kernel(x_pfc, edge_index, W_lin, W_src, W_dst, W_pos, b_pos):
    n, d_in = x_pfc.shape
    e = edge_index.shape[1]
    d_pos = W_pos.shape[0]

    npad = ((n + 1 + NS * 8 - 1) // (NS * 8)) * (NS * 8)
    ch = NW * B
    epad = ((e + ch - 1) // ch) * ch
    ndum = n

    wq = jnp.zeros((d_in, 2), jnp.float32).at[:d_pos, :].set(W_pos)
    wu = W_src + wq
    wv = W_lin - wq
    w16 = jnp.concatenate(
        [wu, wv, wq, jnp.zeros((d_in, TW - 6), jnp.float32)], axis=1)
    b16 = jnp.concatenate([jnp.zeros((2,), jnp.float32), b_pos,
                           jnp.zeros((TW - 4,), jnp.float32)]).reshape(1, TW)
    x_pad = jnp.pad(x_pfc, ((0, npad - n), (0, 0)))
    srcs = edge_index[1]
    dsts = edge_index[0]
    if epad != e:
        pad = jnp.zeros((epad - e,), jnp.int32)
        srcs = jnp.concatenate([srcs, pad])
        dsts = jnp.concatenate([dsts, pad])
    srcs = srcs.reshape(epad // 128, 128)
    dsts = dsts.reshape(epad // 128, 128)
    zeros_acc = jnp.zeros((npad, AW), jnp.float32)

    bn = npad // GRID

    t16 = pl.pallas_call(
        _prologue_body,
        grid=(GRID,),
        in_specs=[
            pl.BlockSpec((bn, d_in), lambda i: (i, 0)),
            pl.BlockSpec((d_in, TW), lambda i: (0, 0)),
            pl.BlockSpec((1, TW), lambda i: (0, 0)),
        ],
        out_specs=pl.BlockSpec((bn, TW), lambda i: (i, 0)),
        out_shape=jax.ShapeDtypeStruct((npad, TW), jnp.float32),
    )(x_pad, w16, b16)

    mesh = plsc.VectorSubcoreMesh(core_axis_name="c", subcore_axis_name="s")
    sc_fn = pl.kernel(
        functools.partial(_sc_edge_body, npad, epad, ndum),
        out_type=jax.ShapeDtypeStruct((NC, npad, AW), jnp.float32),
        mesh=mesh,
        scratch_types=[
            pltpu.VMEM((SUB, 128), jnp.int32),
            pltpu.VMEM((SUB, 128), jnp.int32),
            pltpu.VMEM((SUB, 128), jnp.int32),
            pltpu.VMEM((B, TW), jnp.float32),
            pltpu.VMEM((B, TW), jnp.float32),
            pltpu.VMEM((B, AW), jnp.float32),
            pltpu.VMEM_SHARED((npad, AW), jnp.float32),
            pltpu.SemaphoreType.DMA,
            pltpu.SemaphoreType.DMA,
            pltpu.SemaphoreType.DMA,
            pltpu.SemaphoreType.DMA,
        ],
        compiler_params=pltpu.CompilerParams(
            use_tc_tiling_on_sc=False, needs_layout_passes=False),
    )
    acc = sc_fn(t16, srcs, dsts, zeros_acc)

    out = pl.pallas_call(
        _epilogue_body,
        grid=(GRID,),
        in_specs=[
            pl.BlockSpec((bn, TW), lambda i: (i, 0)),
            pl.BlockSpec((NC, bn, AW), lambda i: (0, i, 0)),
        ],
        out_specs=pl.BlockSpec((bn, 2), lambda i: (i, 0)),
        out_shape=jax.ShapeDtypeStruct((n, 2), jnp.float32),
    )(t16, acc)
    return out
